# Initial kernel scaffold; baseline (speedup 1.0000x reference)
import jax
import jax.numpy as jnp
from jax.experimental import pallas as pl


def kernel(x, weight, bn_gamma, bn_beta, bn_mean, bn_var):
    raise NotImplementedError("write your pallas kernel here")


# speedup vs baseline: 1.7713x; 1.7713x over previous
import functools

import jax
import jax.numpy as jnp
from jax.experimental import pallas as pl
from jax.experimental.pallas import tpu as pltpu


def _mish(y):
    t = jnp.exp(jnp.minimum(y, 20.0))
    u = 1.0 + t
    u2 = u * u
    return y * ((u2 - 1.0) / (u2 + 1.0))


def _conv3x3_bn_mish_kernel(x_ref, w_ref, b_ref, o_ref, *, h, w):
    l = h * w
    c_in = x_ref.shape[0]
    z = jnp.zeros((c_in, 2 * w), jnp.float32)
    xm = jnp.concatenate([z, x_ref[...], z], axis=1)
    lane = jax.lax.broadcasted_iota(jnp.int32, (c_in, l + 2 * w), 1) % w
    acc = jnp.zeros(o_ref.shape, jnp.float32)
    for kw in range(3):
        base = xm[:, w - 1 + kw:(w - 1 + kw) + (l + 2 * w)]
        if kw == 0:
            base = jnp.where(lane != 0, base, 0.0)
        elif kw == 2:
            base = jnp.where(lane != w - 1, base, 0.0)
        for kh in range(3):
            acc = acc + jnp.dot(
                w_ref[kh * 3 + kw],
                base[:, kh * w:kh * w + l],
                preferred_element_type=jnp.float32)
    y = acc + b_ref[...]
    o_ref[...] = _mish(y).astype(o_ref.dtype)


@jax.jit
def _conv_bn_mish(x, weight, bn_gamma, bn_beta, bn_mean, bn_var):
    eps = 1e-5
    n, c_in, h, w = x.shape
    c_out = weight.shape[0]

    scale = bn_gamma / jnp.sqrt(bn_var + eps)
    bias = bn_beta - bn_mean * scale
    w_folded = weight * scale[:, None, None, None]
    w_kk = w_folded.transpose(2, 3, 0, 1).reshape(9, c_out, c_in)
    b_col = bias.reshape(c_out, 1)
    x_flat = x.reshape(n, c_in, h * w)

    out_flat = pl.pallas_call(
        functools.partial(_conv3x3_bn_mish_kernel, h=h, w=w),
        out_shape=jax.ShapeDtypeStruct((n, c_out, h * w), jnp.float32),
        grid_spec=pltpu.PrefetchScalarGridSpec(
            num_scalar_prefetch=0,
            grid=(n,),
            in_specs=[
                pl.BlockSpec((None, c_in, h * w), lambda i: (i, 0, 0)),
                pl.BlockSpec((9, c_out, c_in), lambda i: (0, 0, 0)),
                pl.BlockSpec((c_out, 1), lambda i: (0, 0)),
            ],
            out_specs=pl.BlockSpec((None, c_out, h * w), lambda i: (i, 0, 0)),
        ),
        compiler_params=pltpu.CompilerParams(
            dimension_semantics=("parallel",)),
    )(x_flat, w_kk, b_col)

    return out_flat.reshape(n, c_out, h, w)


def kernel(x, weight, bn_gamma, bn_beta, bn_mean, bn_var):
    return _conv_bn_mish(x, weight, bn_gamma, bn_beta, bn_mean, bn_var)


# speedup vs baseline: 1.7997x; 1.0160x over previous
import functools

import jax
import jax.numpy as jnp
from jax.experimental import pallas as pl
from jax.experimental.pallas import tpu as pltpu


def _mish(y):
    t = jnp.exp(jnp.minimum(y, 20.0))
    u = 1.0 + t
    u2 = u * u
    return y * ((u2 - 1.0) / (u2 + 1.0))


def _conv3x3_bn_mish_kernel(x_ref, w_ref, b_ref, o_ref, *, h, w):
    l = h * w
    rows = x_ref.shape[0]
    z = jnp.zeros((rows, 2 * w), jnp.float32)
    xm = jnp.concatenate([z, x_ref[...], z], axis=1)
    lane = jax.lax.broadcasted_iota(jnp.int32, (rows, l + 2 * w), 1) % w
    acc = jnp.zeros(o_ref.shape, jnp.float32)
    for kw in range(3):
        base = xm[:, w - 1 + kw:(w - 1 + kw) + (l + 2 * w)]
        if kw == 0:
            base = jnp.where(lane != 0, base, 0.0)
        elif kw == 2:
            base = jnp.where(lane != w - 1, base, 0.0)
        for kh in range(3):
            acc = acc + jnp.dot(
                w_ref[kh * 3 + kw],
                base[:, kh * w:kh * w + l],
                preferred_element_type=jnp.float32)
    y = acc + b_ref[...]
    o_ref[...] = _mish(y).astype(o_ref.dtype)


@jax.jit
def _conv_bn_mish(x, weight, bn_gamma, bn_beta, bn_mean, bn_var):
    eps = 1e-5
    n, c_in, h, w = x.shape
    c_out = weight.shape[0]
    g = 2 if n % 2 == 0 else 1

    scale = bn_gamma / jnp.sqrt(bn_var + eps)
    bias = bn_beta - bn_mean * scale
    w_folded = weight * scale[:, None, None, None]
    w_kk = w_folded.transpose(2, 3, 0, 1).reshape(9, c_out, c_in)
    w_blk = jnp.zeros((9, g * c_out, g * c_in), jnp.float32)
    for i in range(g):
        w_blk = w_blk.at[:, i * c_out:(i + 1) * c_out,
                         i * c_in:(i + 1) * c_in].set(w_kk)
    b_col = jnp.tile(bias, (g,)).reshape(g * c_out, 1)

    x2 = x.reshape(n * c_in, h * w)

    out2 = pl.pallas_call(
        functools.partial(_conv3x3_bn_mish_kernel, h=h, w=w),
        out_shape=jax.ShapeDtypeStruct((n * c_out, h * w), jnp.float32),
        grid_spec=pltpu.PrefetchScalarGridSpec(
            num_scalar_prefetch=0,
            grid=(n // g,),
            in_specs=[
                pl.BlockSpec((g * c_in, h * w), lambda i: (i, 0)),
                pl.BlockSpec((9, g * c_out, g * c_in), lambda i: (0, 0, 0)),
                pl.BlockSpec((g * c_out, 1), lambda i: (0, 0)),
            ],
            out_specs=pl.BlockSpec((g * c_out, h * w), lambda i: (i, 0)),
        ),
        compiler_params=pltpu.CompilerParams(
            dimension_semantics=("parallel",)),
    )(x2, w_blk, b_col)

    return out2.reshape(n, c_out, h, w)


def kernel(x, weight, bn_gamma, bn_beta, bn_mean, bn_var):
    return _conv_bn_mish(x, weight, bn_gamma, bn_beta, bn_mean, bn_var)


# speedup vs baseline: 2.6977x; 1.4990x over previous
import functools

import jax
import jax.numpy as jnp
from jax.experimental import pallas as pl
from jax.experimental.pallas import tpu as pltpu


def _mish(y):
    t = jnp.exp(jnp.minimum(y, 20.0))
    u = 1.0 + t
    u2 = u * u
    return y * ((u2 - 1.0) / (u2 + 1.0))


def _conv3x3_bn_mish_kernel(x_ref, w_ref, b_ref, o_ref, *, h, w):
    l = h * w
    rows = x_ref.shape[0]
    z = jnp.zeros((rows, 2 * w), jnp.float32)
    xm = jnp.concatenate([z, x_ref[...], z], axis=1)
    lane = jax.lax.broadcasted_iota(jnp.int32, (rows, l + 2 * w), 1) % w
    bases = []
    for kw in range(3):
        base = xm[:, w - 1 + kw:(w - 1 + kw) + (l + 2 * w)]
        if kw == 0:
            base = jnp.where(lane != 0, base, 0.0)
        elif kw == 2:
            base = jnp.where(lane != w - 1, base, 0.0)
        bases.append(base)
    b3 = jnp.concatenate(bases, axis=0).astype(jnp.bfloat16)
    acc = jnp.zeros(o_ref.shape, jnp.float32)
    for kh in range(3):
        acc = acc + jnp.dot(
            w_ref[kh], b3[:, kh * w:kh * w + l],
            preferred_element_type=jnp.float32)
    y = acc + b_ref[...]
    o_ref[...] = _mish(y).astype(o_ref.dtype)


@jax.jit
def _conv_bn_mish(x, weight, bn_gamma, bn_beta, bn_mean, bn_var):
    eps = 1e-5
    n, c_in, h, w = x.shape
    c_out = weight.shape[0]
    g = 2 if n % 2 == 0 else 1

    scale = bn_gamma / jnp.sqrt(bn_var + eps)
    bias = bn_beta - bn_mean * scale
    w_folded = weight * scale[:, None, None, None]
    w_t = w_folded.transpose(2, 3, 0, 1)
    w3 = jnp.zeros((3, g * c_out, 3 * g * c_in), jnp.float32)
    for kw in range(3):
        for img in range(g):
            w3 = w3.at[:, img * c_out:(img + 1) * c_out,
                       kw * g * c_in + img * c_in:
                       kw * g * c_in + (img + 1) * c_in].set(w_t[:, kw])
    w3 = w3.astype(jnp.bfloat16)
    b_col = jnp.tile(bias, (g,)).reshape(g * c_out, 1)

    x2 = x.reshape(n * c_in, h * w)

    out2 = pl.pallas_call(
        functools.partial(_conv3x3_bn_mish_kernel, h=h, w=w),
        out_shape=jax.ShapeDtypeStruct((n * c_out, h * w), jnp.float32),
        grid_spec=pltpu.PrefetchScalarGridSpec(
            num_scalar_prefetch=0,
            grid=(n // g,),
            in_specs=[
                pl.BlockSpec((g * c_in, h * w), lambda i: (i, 0)),
                pl.BlockSpec((3, g * c_out, 3 * g * c_in), lambda i: (0, 0, 0)),
                pl.BlockSpec((g * c_out, 1), lambda i: (0, 0)),
            ],
            out_specs=pl.BlockSpec((g * c_out, h * w), lambda i: (i, 0)),
        ),
        compiler_params=pltpu.CompilerParams(
            dimension_semantics=("parallel",)),
    )(x2, w3, b_col)

    return out2.reshape(n, c_out, h, w)


def kernel(x, weight, bn_gamma, bn_beta, bn_mean, bn_var):
    return _conv_bn_mish(x, weight, bn_gamma, bn_beta, bn_mean, bn_var)
